# Initial kernel scaffold; baseline (speedup 1.0000x reference)
import jax
import jax.numpy as jnp
from jax.experimental import pallas as pl


def kernel(value, edge_weights, edge_weights_cutoff, edge_index):
    raise NotImplementedError("write your pallas kernel here")


# speedup vs baseline: 9.2365x; 9.2365x over previous
import jax
import jax.numpy as jnp
from jax import lax
from jax.experimental import pallas as pl
from jax.experimental.pallas import tpu as pltpu

N_NODES = 10000
N_EDGES = 320000
NUM_HEADS = 8
D_VALUE = 128
N_PAD = 10240
NB = 2048
EB = 1280
NBLK = N_PAD // NB
EBLK = N_EDGES // EB

_HMAP = [c // 10 for c in range(80)] + [(c - 80) // 6 for c in range(80, 128)]


def _body(value_ref, ew_ref, cut_ref, dst_ref, m_ref, o_ref, num, den):
    n = pl.program_id(0)
    e = pl.program_id(1)

    @pl.when(e == 0)
    def _init():
        num[...] = jnp.zeros_like(num)
        den[...] = jnp.zeros_like(den)

    ewc = cut_ref[...] * ew_ref[...]
    wfull = jnp.dot(jnp.exp(ewc), m_ref[...],
                    preferred_element_type=jnp.float32)
    rows = lax.broadcasted_iota(jnp.int32, (NB, EB), 0) + n * NB
    oh = (rows == dst_ref[...]).astype(jnp.float32)
    num[...] += jnp.dot(oh, wfull * value_ref[...],
                        preferred_element_type=jnp.float32)
    den[...] += jnp.dot(oh, wfull, preferred_element_type=jnp.float32)

    @pl.when(e == EBLK - 1)
    def _finish():
        o_ref[...] = num[...] / (den[...] + 1e-16)


def kernel(value, edge_weights, edge_weights_cutoff, edge_index):
    dst = edge_index[1].reshape(1, N_EDGES)
    cut = edge_weights_cutoff.reshape(N_EDGES, 1)
    hmap = jnp.asarray(_HMAP, dtype=jnp.int32)
    hh = jnp.arange(NUM_HEADS, dtype=jnp.int32)
    m = (hh[:, None] == hmap[None, :]).astype(jnp.float32)

    out = pl.pallas_call(
        _body,
        grid=(NBLK, EBLK),
        in_specs=[
            pl.BlockSpec((EB, D_VALUE), lambda n, e: (e, 0)),
            pl.BlockSpec((EB, NUM_HEADS), lambda n, e: (e, 0)),
            pl.BlockSpec((EB, 1), lambda n, e: (e, 0)),
            pl.BlockSpec((1, EB), lambda n, e: (0, e)),
            pl.BlockSpec((NUM_HEADS, D_VALUE), lambda n, e: (0, 0)),
        ],
        out_specs=pl.BlockSpec((NB, D_VALUE), lambda n, e: (n, 0)),
        out_shape=jax.ShapeDtypeStruct((N_PAD, D_VALUE), jnp.float32),
        scratch_shapes=[
            pltpu.VMEM((NB, D_VALUE), jnp.float32),
            pltpu.VMEM((NB, D_VALUE), jnp.float32),
        ],
    )(value, edge_weights, cut, dst, m)
    return out[:N_NODES]


# speedup vs baseline: 10.0303x; 1.0859x over previous
import jax
import jax.numpy as jnp
from jax import lax
from jax.experimental import pallas as pl
from jax.experimental.pallas import tpu as pltpu

N_NODES = 10000
N_EDGES = 320000
NUM_HEADS = 8
D_VALUE = 128
N_PAD = 10240
NB = 5120
EB = 1280
NBLK = N_PAD // NB
EBLK = N_EDGES // EB

_HMAP = [c // 10 for c in range(80)] + [(c - 80) // 6 for c in range(80, 128)]


def _body(value_ref, ew_ref, cut_ref, dst_ref, m_ref, o_ref, num, den):
    n = pl.program_id(0)
    e = pl.program_id(1)

    @pl.when(e == 0)
    def _init():
        num[...] = jnp.zeros_like(num)
        den[...] = jnp.zeros_like(den)

    ewc = cut_ref[...] * ew_ref[...]
    wfull = jnp.dot(jnp.exp(ewc), m_ref[...],
                    preferred_element_type=jnp.float32)
    rows = lax.broadcasted_iota(jnp.int32, (NB, EB), 0) + n * NB
    oh = (rows == dst_ref[...]).astype(jnp.float32)
    num[...] += jnp.dot(oh, wfull * value_ref[...],
                        preferred_element_type=jnp.float32)
    den[...] += jnp.dot(oh, wfull, preferred_element_type=jnp.float32)

    @pl.when(e == EBLK - 1)
    def _finish():
        o_ref[...] = num[...] / (den[...] + 1e-16)


def kernel(value, edge_weights, edge_weights_cutoff, edge_index):
    dst = edge_index[1].reshape(1, N_EDGES)
    cut = edge_weights_cutoff.reshape(N_EDGES, 1)
    hmap = jnp.asarray(_HMAP, dtype=jnp.int32)
    hh = jnp.arange(NUM_HEADS, dtype=jnp.int32)
    m = (hh[:, None] == hmap[None, :]).astype(jnp.float32)

    out = pl.pallas_call(
        _body,
        grid=(NBLK, EBLK),
        in_specs=[
            pl.BlockSpec((EB, D_VALUE), lambda n, e: (e, 0)),
            pl.BlockSpec((EB, NUM_HEADS), lambda n, e: (e, 0)),
            pl.BlockSpec((EB, 1), lambda n, e: (e, 0)),
            pl.BlockSpec((1, EB), lambda n, e: (0, e)),
            pl.BlockSpec((NUM_HEADS, D_VALUE), lambda n, e: (0, 0)),
        ],
        out_specs=pl.BlockSpec((NB, D_VALUE), lambda n, e: (n, 0)),
        out_shape=jax.ShapeDtypeStruct((N_PAD, D_VALUE), jnp.float32),
        scratch_shapes=[
            pltpu.VMEM((NB, D_VALUE), jnp.float32),
            pltpu.VMEM((NB, D_VALUE), jnp.float32),
        ],
    )(value, edge_weights, cut, dst, m)
    return out[:N_NODES]
